# baseline (speedup 1.0000x reference)
import functools

import jax
import jax.numpy as jnp
from jax import lax
from jax.experimental import pallas as pl
from jax.experimental.pallas import tpu as pltpu
from jax.experimental.pallas import tpu_sc as plsc

_NC = 2
_NS = 16
_NW = _NC * _NS
_L = 16


def _make_sc_kernel(B, PD, RW, NSEG, S):
    mesh = plsc.VectorSubcoreMesh(core_axis_name="c", subcore_axis_name="s")

    @functools.partial(
        pl.kernel,
        mesh=mesh,
        out_type=jax.ShapeDtypeStruct((B, PD), jnp.float32),
        scratch_types=[
            pltpu.VMEM((2, S), jnp.float32),
            pltpu.VMEM((2, RW, S), jnp.float32),
            pltpu.SemaphoreType.DMA,
            pltpu.SemaphoreType.DMA,
            pltpu.SemaphoreType.DMA,
        ],
    )
    def k(x_hbm, e_hbm, o_hbm, e_bufs, bufs, e_sem, in_sem, out_sem):
        wid = lax.axis_index("s") * _NC + lax.axis_index("c")
        rows = pl.ds(wid * RW, RW)

        def e_copy(c, s):
            return pltpu.async_copy(e_hbm.at[pl.ds(c * S, S)], e_bufs.at[s], e_sem)

        def in_copy(c, s):
            return pltpu.async_copy(
                x_hbm.at[rows, pl.ds(c * S, S)], bufs.at[s], in_sem
            )

        def out_copy(c, s):
            return pltpu.async_copy(
                bufs.at[s], o_hbm.at[rows, pl.ds(c * S, S)], out_sem
            )

        def add_block(buf, e_v):
            def body(i, carry):
                sl = pl.ds(i * _L, _L)
                ev = e_v[sl]
                for r in range(RW):
                    plsc.addupdate(buf.at[r, sl], ev)
                return carry

            lax.fori_loop(0, S // _L, body, 0)

        NBUF = 2
        e_d = [None] * NSEG
        in_d = [None] * NSEG
        out_d = [None] * NSEG
        for j in range(min(NBUF - 1, NSEG)):
            e_d[j] = e_copy(j, j % NBUF)
            in_d[j] = in_copy(j, j % NBUF)
        for c in range(NSEG):
            s = c % NBUF
            la = c + NBUF - 1
            if la < NSEG:
                if c >= 1:
                    out_d[c - 1].wait()
                e_d[la] = e_copy(la, la % NBUF)
                in_d[la] = in_copy(la, la % NBUF)
            e_d[c].wait()
            in_d[c].wait()
            add_block(bufs.at[s], e_bufs.at[s])
            out_d[c] = out_copy(c, s)
        for c in range(max(NSEG - NBUF, 0), NSEG):
            out_d[c].wait()

    return k


def kernel(encoded_patches, position_embedding):
    B, P, D = encoded_patches.shape
    PD = P * D
    x2 = encoded_patches.reshape(B, PD)
    e1 = position_embedding.reshape(PD)
    RW = B // _NW
    NSEG = 16
    S = PD // NSEG
    out2 = _make_sc_kernel(B, PD, RW, NSEG, S)(x2, e1)
    return out2.reshape(B, P, D)


# speedup vs baseline: 1.0159x; 1.0159x over previous
import functools

import jax
import jax.numpy as jnp
from jax import lax
from jax.experimental import pallas as pl
from jax.experimental.pallas import tpu as pltpu
from jax.experimental.pallas import tpu_sc as plsc

_NC = 2
_NS = 16
_NW = _NC * _NS
_L = 16


def _make_sc_kernel(B, PD, RW, NSEG, S):
    mesh = plsc.VectorSubcoreMesh(core_axis_name="c", subcore_axis_name="s")

    @functools.partial(
        pl.kernel,
        mesh=mesh,
        out_type=jax.ShapeDtypeStruct((B, PD), jnp.float32),
        scratch_types=[
            pltpu.VMEM((4, S), jnp.float32),
            pltpu.VMEM((4, RW, S), jnp.float32),
            pltpu.SemaphoreType.DMA,
            pltpu.SemaphoreType.DMA,
            pltpu.SemaphoreType.DMA,
        ],
    )
    def k(x_hbm, e_hbm, o_hbm, e_bufs, bufs, e_sem, in_sem, out_sem):
        wid = lax.axis_index("s") * _NC + lax.axis_index("c")
        rows = pl.ds(wid * RW, RW)

        def e_copy(c, s):
            return pltpu.async_copy(e_hbm.at[pl.ds(c * S, S)], e_bufs.at[s], e_sem)

        def in_copy(c, s):
            return pltpu.async_copy(
                x_hbm.at[rows, pl.ds(c * S, S)], bufs.at[s], in_sem
            )

        def out_copy(c, s):
            return pltpu.async_copy(
                bufs.at[s], o_hbm.at[rows, pl.ds(c * S, S)], out_sem
            )

        def add_block(buf, e_v):
            def body(i, carry):
                sl = pl.ds(i * _L, _L)
                ev = e_v[sl]
                for r in range(RW):
                    plsc.addupdate(buf.at[r, sl], ev)
                return carry

            lax.fori_loop(0, S // _L, body, 0)

        NBUF = 4
        e_d = [None] * NSEG
        in_d = [None] * NSEG
        out_d = [None] * NSEG
        for j in range(min(NBUF - 1, NSEG)):
            e_d[j] = e_copy(j, j % NBUF)
            in_d[j] = in_copy(j, j % NBUF)
        for c in range(NSEG):
            s = c % NBUF
            la = c + NBUF - 1
            if la < NSEG:
                if c >= 1:
                    out_d[c - 1].wait()
                e_d[la] = e_copy(la, la % NBUF)
                in_d[la] = in_copy(la, la % NBUF)
            e_d[c].wait()
            in_d[c].wait()
            add_block(bufs.at[s], e_bufs.at[s])
            out_d[c] = out_copy(c, s)
        for c in range(max(NSEG - NBUF, 0), NSEG):
            out_d[c].wait()

    return k


def kernel(encoded_patches, position_embedding):
    B, P, D = encoded_patches.shape
    PD = P * D
    x2 = encoded_patches.reshape(B, PD)
    e1 = position_embedding.reshape(PD)
    RW = B // _NW
    NSEG = 32
    S = PD // NSEG
    out2 = _make_sc_kernel(B, PD, RW, NSEG, S)(x2, e1)
    return out2.reshape(B, P, D)
